# baseline (device time: 30178 ns/iter reference)
import jax
import jax.numpy as jnp
from jax import lax
from jax.experimental import pallas as pl
from jax.experimental.pallas import tpu as pltpu

N_DEV = 4
N_LAYERS = 3


def kernel(x, Win0, Wout0, Win1, Wout1, Win2, Wout2):
    B, D = x.shape
    H = Win0.shape[1]
    R = B // N_DEV

    def body(x_ref, win0_ref, wout0_ref, win1_ref, wout1_ref, win2_ref,
             wout2_ref, out_ref, pa_ref, ra_ref, cb_ref, rb_ref,
             sp_sems, rp_sems, sc_sems, rc_sems):
        my = lax.axis_index("i")
        px = 3 - my
        py = my ^ 1

        wins = [win0_ref, win1_ref, win2_ref]
        wouts = [wout0_ref, wout1_ref, wout2_ref]

        started = []

        def mlp_chunk(xc, l):
            h = jnp.maximum(
                jnp.dot(xc, wins[l][:, :],
                        preferred_element_type=jnp.float32),
                0.0)
            return jnp.dot(h, wouts[l][:, :],
                           preferred_element_type=jnp.float32)

        def rdma(src, dst, ssem, rsem, target):
            return pltpu.make_async_remote_copy(
                src_ref=src, dst_ref=dst, send_sem=ssem, recv_sem=rsem,
                device_id=(target,), device_id_type=pl.DeviceIdType.MESH,
            )

        def send_plain(l, c):
            r = rdma(pa_ref.at[l, c], ra_ref.at[l, c],
                     sp_sems.at[l, c], rp_sems.at[l, c], px)
            r.start()
            return r

        def wait_plain(l, c):
            rdma(pa_ref.at[l, c], ra_ref.at[l, c],
                 rp_sems.at[l, c], rp_sems.at[l, c], px).wait_recv()

        def send_comb(l, c):
            r = rdma(cb_ref.at[l, c], rb_ref.at[l, c],
                     sc_sems.at[l, c], rc_sems.at[l, c], py)
            r.start()
            return r

        def wait_comb(l, c):
            rdma(cb_ref.at[l, c], rb_ref.at[l, c],
                 rc_sems.at[l, c], rc_sems.at[l, c], py).wait_recv()

        for l in (0, 1):
            for c in range(N_DEV):
                if l == 0:
                    xc = x_ref[pl.ds(c * R, R), :]
                else:
                    wait_comb(l - 1, c)
                    xc = cb_ref[l - 1, c, :, :].astype(jnp.float32) + \
                         rb_ref[l - 1, c, :, :].astype(jnp.float32)
                pa_ref[l, c, :, :] = mlp_chunk(xc, l).astype(jnp.bfloat16)
                started.append(send_plain(l, c))
            for c in range(N_DEV):
                wait_plain(l, c)
                cs = pa_ref[l, c, :, :].astype(jnp.float32) + \
                     ra_ref[l, c, :, :].astype(jnp.float32)
                cb_ref[l, c, :, :] = cs.astype(jnp.bfloat16)
                started.append(send_comb(l, c))

        for c in range(N_DEV):
            wait_comb(1, c)
            xc = cb_ref[1, c, :, :].astype(jnp.float32) + \
                 rb_ref[1, c, :, :].astype(jnp.float32)
            pa_ref[2, c, :, :] = mlp_chunk(xc, 2).astype(jnp.bfloat16)

            @pl.when((c != my) & (c != py))
            def _():
                send_plain(2, c)

        wait_plain(2, py)
        cs = pa_ref[2, py, :, :].astype(jnp.float32) + \
             ra_ref[2, py, :, :].astype(jnp.float32)
        cb_ref[2, py, :, :] = cs.astype(jnp.bfloat16)
        started.append(send_comb(2, py))

        wait_plain(2, my)
        cs0 = pa_ref[2, my, :, :].astype(jnp.float32) + \
              ra_ref[2, my, :, :].astype(jnp.float32)
        wait_comb(2, my)
        out_ref[:, :] = cs0 + rb_ref[2, my, :, :].astype(jnp.float32)

        for r in started:
            r.wait_send()
        for c in range(N_DEV):
            @pl.when((c != my) & (c != py))
            def _():
                rdma(pa_ref.at[2, c], ra_ref.at[2, c],
                     sp_sems.at[2, c], rp_sems.at[2, c], px).wait_send()

    return pl.pallas_call(
        body,
        out_shape=jax.ShapeDtypeStruct((R, D), jnp.float32),
        in_specs=[pl.BlockSpec(memory_space=pltpu.VMEM)] * 7,
        out_specs=pl.BlockSpec(memory_space=pltpu.VMEM),
        scratch_shapes=[
            pltpu.VMEM((N_LAYERS, N_DEV, R, D), jnp.bfloat16),
            pltpu.VMEM((N_LAYERS, N_DEV, R, D), jnp.bfloat16),
            pltpu.VMEM((N_LAYERS, N_DEV, R, D), jnp.bfloat16),
            pltpu.VMEM((N_LAYERS, N_DEV, R, D), jnp.bfloat16),
            pltpu.SemaphoreType.DMA((N_LAYERS, N_DEV)),
            pltpu.SemaphoreType.DMA((N_LAYERS, N_DEV)),
            pltpu.SemaphoreType.DMA((N_LAYERS, N_DEV)),
            pltpu.SemaphoreType.DMA((N_LAYERS, N_DEV)),
        ],
    )(x, Win0, Wout0, Win1, Wout1, Win2, Wout2)


# device time: 26996 ns/iter; 1.1179x vs baseline; 1.1179x over previous
import jax
import jax.numpy as jnp
from jax import lax
from jax.experimental import pallas as pl
from jax.experimental.pallas import tpu as pltpu

N_DEV = 4
N_LAYERS = 3


def kernel(x, Win0, Wout0, Win1, Wout1, Win2, Wout2):
    B, D = x.shape
    H = Win0.shape[1]
    R = B // N_DEV

    def body(x_ref, win0_ref, wout0_ref, win1_ref, wout1_ref, win2_ref,
             wout2_ref, out_ref, bc_ref, part_ref, rs_ref,
             send_b, recv_b, send_rs, recv_rs):
        my = lax.axis_index("i")
        wins = [win0_ref, win1_ref, win2_ref]
        wouts = [wout0_ref, wout1_ref, wout2_ref]

        barrier_sem = pltpu.get_barrier_semaphore()
        for o in (1, 2, 3):
            pl.semaphore_signal(barrier_sem, inc=1,
                                device_id=((my + o) % N_DEV,),
                                device_id_type=pl.DeviceIdType.MESH)
        pl.semaphore_wait(barrier_sem, 3)

        started = []

        def mlp_chunk(xc, l):
            h = jnp.maximum(
                jnp.dot(xc, wins[l][:, :],
                        preferred_element_type=jnp.float32),
                0.0)
            return jnp.dot(h, wouts[l][:, :],
                           preferred_element_type=jnp.float32)

        def bcast_chunk(l, c):
            src = bc_ref.at[l, my, pl.ds(c * R, R), :]
            for o in (2, 1, 3):
                e = (my + o) % N_DEV
                rdma = pltpu.make_async_remote_copy(
                    src_ref=src, dst_ref=src,
                    send_sem=send_b.at[l, c, o - 1],
                    recv_sem=recv_b.at[l, my, c],
                    device_id=(e,), device_id_type=pl.DeviceIdType.MESH,
                )
                rdma.start()
                started.append(rdma)

        def gather_chunk(l, c):
            acc = bc_ref[l, my, pl.ds(c * R, R), :].astype(jnp.float32)
            for o in (1, 3, 2):
                s = (my + o) % N_DEV
                pltpu.make_async_remote_copy(
                    src_ref=bc_ref.at[l, s, pl.ds(c * R, R), :],
                    dst_ref=bc_ref.at[l, s, pl.ds(c * R, R), :],
                    send_sem=send_b.at[l, c, 0],
                    recv_sem=recv_b.at[l, s, c],
                    device_id=(s,), device_id_type=pl.DeviceIdType.MESH,
                ).wait_recv()
                acc = acc + bc_ref[l, s, pl.ds(c * R, R), :].astype(jnp.float32)
            return acc

        def rs_send(c):
            return pltpu.make_async_remote_copy(
                src_ref=part_ref.at[c],
                dst_ref=rs_ref.at[my],
                send_sem=send_rs.at[c],
                recv_sem=recv_rs.at[my],
                device_id=(c,), device_id_type=pl.DeviceIdType.MESH,
            )

        for l in (0, 1):
            for c in range(N_DEV):
                if l == 0:
                    xc = x_ref[pl.ds(c * R, R), :]
                else:
                    xc = gather_chunk(0, c)
                bc_ref[l, my, pl.ds(c * R, R), :] = \
                    mlp_chunk(xc, l).astype(jnp.bfloat16)
                bcast_chunk(l, c)

        for c in range(N_DEV):
            xc = gather_chunk(1, c)
            part_ref[c, :, :] = mlp_chunk(xc, 2).astype(jnp.bfloat16)

            @pl.when(c != my)
            def _():
                rs_send(c).start()

        acc = part_ref[my, :, :].astype(jnp.float32)
        for o in (1, 3, 2):
            s = (my + o) % N_DEV
            pltpu.make_async_remote_copy(
                src_ref=rs_ref.at[s], dst_ref=rs_ref.at[s],
                send_sem=send_rs.at[0], recv_sem=recv_rs.at[s],
                device_id=(s,), device_id_type=pl.DeviceIdType.MESH,
            ).wait_recv()
            acc = acc + rs_ref[s, :, :].astype(jnp.float32)
        out_ref[:, :] = acc

        for rdma in started:
            rdma.wait_send()
        for c in range(N_DEV):
            @pl.when(c != my)
            def _():
                rs_send(c).wait_send()

    return pl.pallas_call(
        body,
        out_shape=jax.ShapeDtypeStruct((R, D), jnp.float32),
        in_specs=[pl.BlockSpec(memory_space=pltpu.VMEM)] * 7,
        out_specs=pl.BlockSpec(memory_space=pltpu.VMEM),
        scratch_shapes=[
            pltpu.VMEM((2, N_DEV, B, D), jnp.bfloat16),
            pltpu.VMEM((N_DEV, R, D), jnp.bfloat16),
            pltpu.VMEM((N_DEV, R, D), jnp.bfloat16),
            pltpu.SemaphoreType.DMA((2, N_DEV, N_DEV - 1)),
            pltpu.SemaphoreType.DMA((2, N_DEV, N_DEV)),
            pltpu.SemaphoreType.DMA((N_DEV,)),
            pltpu.SemaphoreType.DMA((N_DEV,)),
        ],
        compiler_params=pltpu.CompilerParams(collective_id=0),
    )(x, Win0, Wout0, Win1, Wout1, Win2, Wout2)
